# Initial kernel scaffold; baseline (speedup 1.0000x reference)
import jax
import jax.numpy as jnp
from jax.experimental import pallas as pl


def kernel(x, conv_w, bn_gamma, bn_beta, bn_mean, bn_var, l1_conv1_w, l1_bn1_gamma, l1_bn1_beta, l1_bn1_mean, l1_bn1_var, l1_conv2_w, l1_bn2_gamma, l1_bn2_beta, l1_bn2_mean, l1_bn2_var, l2_conv1_w, l2_bn1_gamma, l2_bn1_beta, l2_bn1_mean, l2_bn1_var, l2_conv2_w, l2_bn2_gamma, l2_bn2_beta, l2_bn2_mean, l2_bn2_var, l2_ds_w, l2_ds_bn_gamma, l2_ds_bn_beta, l2_ds_bn_mean, l2_ds_bn_var, l3_conv1_w, l3_bn1_gamma, l3_bn1_beta, l3_bn1_mean, l3_bn1_var, l3_conv2_w, l3_bn2_gamma, l3_bn2_beta, l3_bn2_mean, l3_bn2_var, l3_ds_w, l3_ds_bn_gamma, l3_ds_bn_beta, l3_ds_bn_mean, l3_ds_bn_var, fc_w, fc_b):
    raise NotImplementedError("write your pallas kernel here")


# speedup vs baseline: 2.8656x; 2.8656x over previous
import functools

import jax
import jax.numpy as jnp
from jax.experimental import pallas as pl
from jax.experimental.pallas import tpu as pltpu

_EPS = 1e-5
_BF = jnp.bfloat16


def _fold_bn(w_oihw, gamma, beta, mean, var):
    c_out, c_in = w_oihw.shape[0], w_oihw.shape[1]
    scale = gamma / jnp.sqrt(var + _EPS)
    w = jnp.transpose(w_oihw, (2, 3, 1, 0)).reshape(9, c_in, c_out)
    w = (w * scale[None, None, :]).astype(_BF)
    b = (beta - mean * scale).astype(jnp.float32)[None, :]
    return w, b


def _pick_tm(m):
    for tm in (2048, 1024, 512, 256, 128):
        if m % tm == 0:
            return tm
    raise ValueError(f"row count {m} not divisible by any tile size")


def _stage1_body(xm_ref, xh_ref, ws_ref, bs_ref, w1_ref, b1_ref, w2_ref,
                 b2_ref, o_ref, win_ref, a_ref, b_ref,
                 *, m, halo, wp):
    offs = tuple((k // 3) * wp + (k % 3) for k in range(9))
    q = jax.lax.broadcasted_iota(jnp.int32, (m, 1), 0) % (wp * wp)
    h, w = q // wp, q % wp
    mask = ((h >= 1) & (h < wp - 1) & (w >= 1) & (w < wp - 1)
            ).astype(jnp.float32)

    def conv9(src, w_ref):
        acc = jnp.zeros((m, w_ref.shape[2]), jnp.float32)
        for k in range(9):
            acc = acc + jnp.dot(src(offs[k]), w_ref[k],
                                preferred_element_type=jnp.float32)
        return acc

    def stash(dst_ref, val):
        dst_ref[pl.ds(0, 40), :] = jnp.zeros((40, 16), _BF)
        dst_ref[pl.ds(m + 32, 40), :] = jnp.zeros((40, 16), _BF)
        dst_ref[pl.ds(35, m), :] = val.astype(_BF)

    win_ref[pl.ds(0, m), :] = xm_ref[...]
    win_ref[pl.ds(m, halo), :] = xh_ref[...]

    stem = jnp.maximum(conv9(lambda o: win_ref[pl.ds(o, m), :], ws_ref)
                       + bs_ref[...], 0.0) * mask
    stash(a_ref, stem)

    h1 = jnp.maximum(conv9(lambda o: a_ref[pl.ds(o, m), :], w1_ref)
                     + b1_ref[...], 0.0) * mask
    stash(b_ref, h1)

    out = conv9(lambda o: b_ref[pl.ds(o, m), :], w2_ref) + b2_ref[...] + stem
    o_ref[...] = (jnp.maximum(out, 0.0) * mask).astype(o_ref.dtype)


def _stage1(xg, ws, bs, w1, b1, w2, b2, n):
    wp = 34
    b_imgs = 8 if n % 8 == 0 else 2
    m = b_imgs * wp * wp
    halo = 136
    assert m % halo == 0
    grid = (n * wp * wp) // m
    ratio = m // halo
    body = functools.partial(_stage1_body, m=m, halo=halo, wp=wp)
    return pl.pallas_call(
        body,
        out_shape=jax.ShapeDtypeStruct((n * wp * wp, 16), _BF),
        grid=(grid,),
        in_specs=[
            pl.BlockSpec((m, 3), lambda t: (t, 0)),
            pl.BlockSpec((halo, 3), lambda t: ((t + 1) * ratio, 0)),
            pl.BlockSpec((9, 3, 16), lambda t: (0, 0, 0)),
            pl.BlockSpec((1, 16), lambda t: (0, 0)),
            pl.BlockSpec((9, 16, 16), lambda t: (0, 0, 0)),
            pl.BlockSpec((1, 16), lambda t: (0, 0)),
            pl.BlockSpec((9, 16, 16), lambda t: (0, 0, 0)),
            pl.BlockSpec((1, 16), lambda t: (0, 0)),
        ],
        out_specs=pl.BlockSpec((m, 16), lambda t: (t, 0)),
        scratch_shapes=[
            pltpu.VMEM((m + halo, 3), _BF),
            pltpu.VMEM((m + 72, 16), _BF),
            pltpu.VMEM((m + 72, 16), _BF),
        ],
        compiler_params=pltpu.CompilerParams(
            dimension_semantics=("parallel",)),
    )(xg, xg, ws, bs, w1, b1, w2, b2)


def _conv_body(*refs, wp, tm, halo, relu_to, has_res, has_mask):
    it = iter(refs)
    xm_ref, xh_ref, w_ref, b_ref = next(it), next(it), next(it), next(it)
    res_ref = next(it) if has_res else None
    o_ref, win_ref = next(it), next(it)
    c_out = w_ref.shape[2]

    win_ref[pl.ds(0, tm), :] = xm_ref[...]
    win_ref[pl.ds(tm, halo), :] = xh_ref[...]

    acc = jnp.zeros((tm, c_out), jnp.float32)
    for k in range(9):
        off = (k // 3) * wp + (k % 3)
        acc = acc + jnp.dot(win_ref[pl.ds(off, tm), :], w_ref[k],
                            preferred_element_type=jnp.float32)
    out = acc + b_ref[...]
    if has_res:
        out = out + res_ref[...].astype(jnp.float32)
    if relu_to == c_out:
        out = jnp.maximum(out, 0.0)
    elif relu_to:
        lane = jax.lax.broadcasted_iota(jnp.int32, (1, c_out), 1)
        out = jnp.where(lane < relu_to, jnp.maximum(out, 0.0), out)
    if has_mask:
        r = (pl.program_id(0) * tm
             + jax.lax.broadcasted_iota(jnp.int32, (tm, 1), 0)) % (wp * wp)
        h, w = r // wp, r % wp
        out = out * ((h >= 1) & (h < wp - 1) & (w >= 1) & (w < wp - 1)
                     ).astype(jnp.float32)
    o_ref[...] = out.astype(o_ref.dtype)


def _conv(x_flat, w, b, *, wp, relu_to, res=None, mask=False):
    m_full, c_in = x_flat.shape
    c_out = w.shape[2]
    tm = _pick_tm(m_full)
    halo = 128
    guard = wp + 1
    xg = jnp.pad(x_flat, ((guard, halo - guard), (0, 0)))
    ratio = tm // halo
    args = [xg, xg, w, b]
    in_specs = [
        pl.BlockSpec((tm, c_in), lambda t: (t, 0)),
        pl.BlockSpec((halo, c_in), lambda t: ((t + 1) * ratio, 0)),
        pl.BlockSpec((9, c_in, c_out), lambda t: (0, 0, 0)),
        pl.BlockSpec((1, c_out), lambda t: (0, 0)),
    ]
    if res is not None:
        args.append(res)
        in_specs.append(pl.BlockSpec((tm, c_out), lambda t: (t, 0)))
    body = functools.partial(_conv_body, wp=wp, tm=tm, halo=halo,
                             relu_to=relu_to, has_res=res is not None,
                             has_mask=mask)
    return pl.pallas_call(
        body,
        out_shape=jax.ShapeDtypeStruct((m_full, c_out), _BF),
        grid=(m_full // tm,),
        in_specs=in_specs,
        out_specs=pl.BlockSpec((tm, c_out), lambda t: (t, 0)),
        scratch_shapes=[pltpu.VMEM((tm + halo, c_in), _BF)],
        compiler_params=pltpu.CompilerParams(
            dimension_semantics=("parallel",)),
    )(*args)


def _pool_fc_body(x_ref, w_ref, b_ref, o_ref):
    pooled = jnp.mean(x_ref[...].astype(jnp.float32), axis=1)
    o_ref[...] = (jnp.dot(pooled, w_ref[...],
                          preferred_element_type=jnp.float32) + b_ref[...])


def _pool_fc(feats, fc_w, fc_b):
    n, p, c = feats.shape
    k = fc_w.shape[1]
    bt = 128 if n % 128 == 0 else n
    return pl.pallas_call(
        _pool_fc_body,
        out_shape=jax.ShapeDtypeStruct((n, k), jnp.float32),
        grid=(n // bt,),
        in_specs=[
            pl.BlockSpec((bt, p, c), lambda t: (t, 0, 0)),
            pl.BlockSpec((c, k), lambda t: (0, 0)),
            pl.BlockSpec((1, k), lambda t: (0, 0)),
        ],
        out_specs=pl.BlockSpec((bt, k), lambda t: (t, 0)),
        compiler_params=pltpu.CompilerParams(
            dimension_semantics=("parallel",)),
    )(feats, fc_w.astype(jnp.float32), fc_b[None, :].astype(jnp.float32))


def _subsample_pad(y_flat, n, g, c_pair):
    go = (g - 2) // 2 + 2
    t = y_flat.reshape(n, g, g, 2 * c_pair)[:, 1:g - 1:2, 1:g - 1:2, :]
    pad = ((0, 0), (1, 1), (1, 1), (0, 0))
    c1 = jnp.pad(t[..., :c_pair], pad).reshape(n * go * go, c_pair)
    ds = jnp.pad(t[..., c_pair:], pad).reshape(n * go * go, c_pair)
    return c1, ds


def kernel(x, conv_w, bn_gamma, bn_beta, bn_mean, bn_var, l1_conv1_w, l1_bn1_gamma, l1_bn1_beta, l1_bn1_mean, l1_bn1_var, l1_conv2_w, l1_bn2_gamma, l1_bn2_beta, l1_bn2_mean, l1_bn2_var, l2_conv1_w, l2_bn1_gamma, l2_bn1_beta, l2_bn1_mean, l2_bn1_var, l2_conv2_w, l2_bn2_gamma, l2_bn2_beta, l2_bn2_mean, l2_bn2_var, l2_ds_w, l2_ds_bn_gamma, l2_ds_bn_beta, l2_ds_bn_mean, l2_ds_bn_var, l3_conv1_w, l3_bn1_gamma, l3_bn1_beta, l3_bn1_mean, l3_bn1_var, l3_conv2_w, l3_bn2_gamma, l3_bn2_beta, l3_bn2_mean, l3_bn2_var, l3_ds_w, l3_ds_bn_gamma, l3_ds_bn_beta, l3_ds_bn_mean, l3_ds_bn_var, fc_w, fc_b):
    n = x.shape[0]

    ws, bs = _fold_bn(conv_w, bn_gamma, bn_beta, bn_mean, bn_var)
    w11, b11 = _fold_bn(l1_conv1_w, l1_bn1_gamma, l1_bn1_beta, l1_bn1_mean,
                        l1_bn1_var)
    w12, b12 = _fold_bn(l1_conv2_w, l1_bn2_gamma, l1_bn2_beta, l1_bn2_mean,
                        l1_bn2_var)
    w21, b21 = _fold_bn(l2_conv1_w, l2_bn1_gamma, l2_bn1_beta, l2_bn1_mean,
                        l2_bn1_var)
    w22, b22 = _fold_bn(l2_conv2_w, l2_bn2_gamma, l2_bn2_beta, l2_bn2_mean,
                        l2_bn2_var)
    w2d, b2d = _fold_bn(l2_ds_w, l2_ds_bn_gamma, l2_ds_bn_beta, l2_ds_bn_mean,
                        l2_ds_bn_var)
    w31, b31 = _fold_bn(l3_conv1_w, l3_bn1_gamma, l3_bn1_beta, l3_bn1_mean,
                        l3_bn1_var)
    w32, b32 = _fold_bn(l3_conv2_w, l3_bn2_gamma, l3_bn2_beta, l3_bn2_mean,
                        l3_bn2_var)
    w3d, b3d = _fold_bn(l3_ds_w, l3_ds_bn_gamma, l3_ds_bn_beta, l3_ds_bn_mean,
                        l3_ds_bn_var)
    w2a = jnp.concatenate([w21, w2d], axis=2)
    b2a = jnp.concatenate([b21, b2d], axis=1)
    w3a = jnp.concatenate([w31, w3d], axis=2)
    b3a = jnp.concatenate([b31, b3d], axis=1)

    xh = jnp.transpose(x, (0, 2, 3, 1))
    xp = jnp.pad(xh, ((0, 0), (1, 1), (1, 1), (0, 0)))
    xg = jnp.pad(xp.reshape(n * 34 * 34, 3).astype(_BF), ((35, 101), (0, 0)))

    y1 = _stage1(xg, ws, bs, w11, b11, w12, b12, n)

    y2 = _conv(y1, w2a, b2a, wp=34, relu_to=32)
    c1, ds = _subsample_pad(y2, n, 34, 32)
    y3 = _conv(c1, w22, b22, wp=18, relu_to=32, res=ds, mask=True)

    y4 = _conv(y3, w3a, b3a, wp=18, relu_to=64)
    c1, ds = _subsample_pad(y4, n, 18, 64)
    y5 = _conv(c1, w32, b32, wp=10, relu_to=64, res=ds)

    feats = y5.reshape(n, 10, 10, 64)[:, 1:9, 1:9, :].reshape(n, 64, 64)
    return _pool_fc(feats, fc_w, fc_b)


# speedup vs baseline: 3.2251x; 1.1254x over previous
import functools

import jax
import jax.numpy as jnp
from jax.experimental import pallas as pl
from jax.experimental.pallas import tpu as pltpu

_EPS = 1e-5
_BF = jnp.bfloat16


def _fold_bn(w_oihw, gamma, beta, mean, var):
    c_out, c_in = w_oihw.shape[0], w_oihw.shape[1]
    scale = gamma / jnp.sqrt(var + _EPS)
    w = jnp.transpose(w_oihw, (2, 3, 1, 0)).reshape(9, c_in, c_out)
    w = w * scale[None, None, :]
    b = (beta - mean * scale).astype(jnp.float32)[None, :]
    return w, b


def _pick_tm(m):
    for tm in (2048, 1024, 512, 256, 128):
        if m % tm == 0:
            return tm
    raise ValueError(f"row count {m} not divisible by any tile size")


_WP1 = 34
_PIX = 1156
_GRP = 145


def _pack_taps(w_taps, bias, wp, p, cin, cout):
    js = sorted({(s + (k // 3) * wp + (k % 3) - (wp + 1)) // p
                 for k in range(9) for s in range(p)})
    mats = {j: jnp.zeros((p * cin, p * cout), jnp.float32) for j in js}
    for k in range(9):
        d = (k // 3) * wp + (k % 3) - (wp + 1)
        for s in range(p):
            j, si = (s + d) // p, (s + d) % p
            mats[j] = mats[j].at[si * cin:(si + 1) * cin,
                                 s * cout:(s + 1) * cout].add(w_taps[k])
    wpk = jnp.stack([mats[j] for j in js]).astype(_BF)
    bpk = jnp.tile(bias, (1, p)).astype(jnp.float32)
    return wpk, bpk, tuple(js)


def _stage1_body(xm_ref, xh_ref, ws_ref, bs_ref, w1_ref, b1_ref, w2_ref,
                 b2_ref, o_ref, win_ref, a_ref, b_ref, *, mg, halo, js):
    r = jax.lax.broadcasted_iota(jnp.int32, (mg, 128), 0) % _GRP
    lane = jax.lax.broadcasted_iota(jnp.int32, (mg, 128), 1)
    p = 8 * r + lane // 16
    h, w = p // _WP1, p % _WP1
    mask = ((h >= 1) & (h < _WP1 - 1) & (w >= 1) & (w < _WP1 - 1)
            ).astype(jnp.float32)

    def conv(src_ref, w_ref):
        acc = jnp.zeros((mg, 128), jnp.float32)
        for i, j in enumerate(js):
            acc = acc + jnp.dot(src_ref[pl.ds(8 + j, mg), :], w_ref[i],
                                preferred_element_type=jnp.float32)
        return acc

    def stash(dst_ref, val):
        dst_ref[pl.ds(0, 8), :] = jnp.zeros((8, 128), _BF)
        dst_ref[pl.ds(mg + 8, 8), :] = jnp.zeros((8, 128), _BF)
        dst_ref[pl.ds(8, mg), :] = val.astype(_BF)

    win_ref[pl.ds(0, mg), :] = xm_ref[...]
    win_ref[pl.ds(mg, halo), :] = xh_ref[...]

    stem = jnp.maximum(conv(win_ref, ws_ref) + bs_ref[...], 0.0) * mask
    stash(a_ref, stem)
    h1 = jnp.maximum(conv(a_ref, w1_ref) + b1_ref[...], 0.0) * mask
    stash(b_ref, h1)
    out = conv(b_ref, w2_ref) + b2_ref[...] + stem
    o_ref[...] = (jnp.maximum(out, 0.0) * mask).astype(o_ref.dtype)


def _stage1(xg, ws, bs, w1, b1, w2, b2, n, js):
    b_imgs = 8 if n % 8 == 0 else 2
    mg = b_imgs * _GRP
    halo = 40
    assert mg % halo == 0
    grid = (n * _GRP) // mg
    ratio = mg // halo
    nj = len(js)
    body = functools.partial(_stage1_body, mg=mg, halo=halo, js=js)
    return pl.pallas_call(
        body,
        out_shape=jax.ShapeDtypeStruct((n * _GRP, 128), _BF),
        grid=(grid,),
        in_specs=[
            pl.BlockSpec((mg, 24), lambda t: (t, 0)),
            pl.BlockSpec((halo, 24), lambda t: ((t + 1) * ratio, 0)),
            pl.BlockSpec((nj, 24, 128), lambda t: (0, 0, 0)),
            pl.BlockSpec((1, 128), lambda t: (0, 0)),
            pl.BlockSpec((nj, 128, 128), lambda t: (0, 0, 0)),
            pl.BlockSpec((1, 128), lambda t: (0, 0)),
            pl.BlockSpec((nj, 128, 128), lambda t: (0, 0, 0)),
            pl.BlockSpec((1, 128), lambda t: (0, 0)),
        ],
        out_specs=pl.BlockSpec((mg, 128), lambda t: (t, 0)),
        scratch_shapes=[
            pltpu.VMEM((mg + halo, 24), _BF),
            pltpu.VMEM((mg + 16, 128), _BF),
            pltpu.VMEM((mg + 16, 128), _BF),
        ],
        compiler_params=pltpu.CompilerParams(
            dimension_semantics=("parallel",)),
    )(xg, xg, ws, bs, w1, b1, w2, b2)


def _conv_body(*refs, wp, tm, halo, relu_to, has_res, has_mask):
    it = iter(refs)
    xm_ref, xh_ref, w_ref, b_ref = next(it), next(it), next(it), next(it)
    res_ref = next(it) if has_res else None
    o_ref, win_ref = next(it), next(it)
    c_out = w_ref.shape[2]

    win_ref[pl.ds(0, tm), :] = xm_ref[...]
    win_ref[pl.ds(tm, halo), :] = xh_ref[...]

    acc = jnp.zeros((tm, c_out), jnp.float32)
    for k in range(9):
        off = (k // 3) * wp + (k % 3)
        acc = acc + jnp.dot(win_ref[pl.ds(off, tm), :], w_ref[k],
                            preferred_element_type=jnp.float32)
    out = acc + b_ref[...]
    if has_res:
        out = out + res_ref[...].astype(jnp.float32)
    if relu_to == c_out:
        out = jnp.maximum(out, 0.0)
    elif relu_to:
        lane = jax.lax.broadcasted_iota(jnp.int32, (1, c_out), 1)
        out = jnp.where(lane < relu_to, jnp.maximum(out, 0.0), out)
    if has_mask:
        r = (pl.program_id(0) * tm
             + jax.lax.broadcasted_iota(jnp.int32, (tm, 1), 0)) % (wp * wp)
        h, w = r // wp, r % wp
        out = out * ((h >= 1) & (h < wp - 1) & (w >= 1) & (w < wp - 1)
                     ).astype(jnp.float32)
    o_ref[...] = out.astype(o_ref.dtype)


def _conv(x_flat, w, b, *, wp, relu_to, res=None, mask=False):
    m_full, c_in = x_flat.shape
    c_out = w.shape[2]
    w = w.astype(_BF)
    tm = _pick_tm(m_full)
    halo = 128
    guard = wp + 1
    xg = jnp.pad(x_flat, ((guard, halo - guard), (0, 0)))
    ratio = tm // halo
    args = [xg, xg, w, b]
    in_specs = [
        pl.BlockSpec((tm, c_in), lambda t: (t, 0)),
        pl.BlockSpec((halo, c_in), lambda t: ((t + 1) * ratio, 0)),
        pl.BlockSpec((9, c_in, c_out), lambda t: (0, 0, 0)),
        pl.BlockSpec((1, c_out), lambda t: (0, 0)),
    ]
    if res is not None:
        args.append(res)
        in_specs.append(pl.BlockSpec((tm, c_out), lambda t: (t, 0)))
    body = functools.partial(_conv_body, wp=wp, tm=tm, halo=halo,
                             relu_to=relu_to, has_res=res is not None,
                             has_mask=mask)
    return pl.pallas_call(
        body,
        out_shape=jax.ShapeDtypeStruct((m_full, c_out), _BF),
        grid=(m_full // tm,),
        in_specs=in_specs,
        out_specs=pl.BlockSpec((tm, c_out), lambda t: (t, 0)),
        scratch_shapes=[pltpu.VMEM((tm + halo, c_in), _BF)],
        compiler_params=pltpu.CompilerParams(
            dimension_semantics=("parallel",)),
    )(*args)


def _pool_fc_body(x_ref, w_ref, b_ref, o_ref):
    pooled = jnp.mean(x_ref[...].astype(jnp.float32), axis=1)
    o_ref[...] = (jnp.dot(pooled, w_ref[...],
                          preferred_element_type=jnp.float32) + b_ref[...])


def _pool_fc(feats, fc_w, fc_b):
    n, p, c = feats.shape
    k = fc_w.shape[1]
    bt = 128 if n % 128 == 0 else n
    return pl.pallas_call(
        _pool_fc_body,
        out_shape=jax.ShapeDtypeStruct((n, k), jnp.float32),
        grid=(n // bt,),
        in_specs=[
            pl.BlockSpec((bt, p, c), lambda t: (t, 0, 0)),
            pl.BlockSpec((c, k), lambda t: (0, 0)),
            pl.BlockSpec((1, k), lambda t: (0, 0)),
        ],
        out_specs=pl.BlockSpec((bt, k), lambda t: (t, 0)),
        compiler_params=pltpu.CompilerParams(
            dimension_semantics=("parallel",)),
    )(feats, fc_w.astype(jnp.float32), fc_b[None, :].astype(jnp.float32))


def _subsample_pad(y_flat, n, g, c_pair):
    go = (g - 2) // 2 + 2
    t = y_flat.reshape(n, g, g, 2 * c_pair)[:, 1:g - 1:2, 1:g - 1:2, :]
    pad = ((0, 0), (1, 1), (1, 1), (0, 0))
    c1 = jnp.pad(t[..., :c_pair], pad).reshape(n * go * go, c_pair)
    ds = jnp.pad(t[..., c_pair:], pad).reshape(n * go * go, c_pair)
    return c1, ds


def kernel(x, conv_w, bn_gamma, bn_beta, bn_mean, bn_var, l1_conv1_w, l1_bn1_gamma, l1_bn1_beta, l1_bn1_mean, l1_bn1_var, l1_conv2_w, l1_bn2_gamma, l1_bn2_beta, l1_bn2_mean, l1_bn2_var, l2_conv1_w, l2_bn1_gamma, l2_bn1_beta, l2_bn1_mean, l2_bn1_var, l2_conv2_w, l2_bn2_gamma, l2_bn2_beta, l2_bn2_mean, l2_bn2_var, l2_ds_w, l2_ds_bn_gamma, l2_ds_bn_beta, l2_ds_bn_mean, l2_ds_bn_var, l3_conv1_w, l3_bn1_gamma, l3_bn1_beta, l3_bn1_mean, l3_bn1_var, l3_conv2_w, l3_bn2_gamma, l3_bn2_beta, l3_bn2_mean, l3_bn2_var, l3_ds_w, l3_ds_bn_gamma, l3_ds_bn_beta, l3_ds_bn_mean, l3_ds_bn_var, fc_w, fc_b):
    n = x.shape[0]

    ws, bs = _fold_bn(conv_w, bn_gamma, bn_beta, bn_mean, bn_var)
    w11, b11 = _fold_bn(l1_conv1_w, l1_bn1_gamma, l1_bn1_beta, l1_bn1_mean,
                        l1_bn1_var)
    w12, b12 = _fold_bn(l1_conv2_w, l1_bn2_gamma, l1_bn2_beta, l1_bn2_mean,
                        l1_bn2_var)
    w21, b21 = _fold_bn(l2_conv1_w, l2_bn1_gamma, l2_bn1_beta, l2_bn1_mean,
                        l2_bn1_var)
    w22, b22 = _fold_bn(l2_conv2_w, l2_bn2_gamma, l2_bn2_beta, l2_bn2_mean,
                        l2_bn2_var)
    w2d, b2d = _fold_bn(l2_ds_w, l2_ds_bn_gamma, l2_ds_bn_beta, l2_ds_bn_mean,
                        l2_ds_bn_var)
    w31, b31 = _fold_bn(l3_conv1_w, l3_bn1_gamma, l3_bn1_beta, l3_bn1_mean,
                        l3_bn1_var)
    w32, b32 = _fold_bn(l3_conv2_w, l3_bn2_gamma, l3_bn2_beta, l3_bn2_mean,
                        l3_bn2_var)
    w3d, b3d = _fold_bn(l3_ds_w, l3_ds_bn_gamma, l3_ds_bn_beta, l3_ds_bn_mean,
                        l3_ds_bn_var)
    w2a = jnp.concatenate([w21, w2d], axis=2)
    b2a = jnp.concatenate([b21, b2d], axis=1)
    w3a = jnp.concatenate([w31, w3d], axis=2)
    b3a = jnp.concatenate([b31, b3d], axis=1)

    wsp, bsp, js = _pack_taps(ws, bs, _WP1, 8, 3, 16)
    w1p, b1p, _ = _pack_taps(w11, b11, _WP1, 8, 16, 16)
    w2p, b2p, _ = _pack_taps(w12, b12, _WP1, 8, 16, 16)

    xh = jnp.transpose(x, (0, 2, 3, 1))
    xp = jnp.pad(xh, ((0, 0), (1, 1), (1, 1), (0, 0))).reshape(n, _PIX, 3)
    xp = jnp.pad(xp, ((0, 0), (0, 4), (0, 0))).reshape(n * _GRP, 24)
    xg = jnp.pad(xp.astype(_BF), ((8, 32), (0, 0)))

    y1p = _stage1(xg, wsp, bsp, w1p, b1p, w2p, b2p, n, js)
    y1 = y1p.reshape(n, _GRP * 8, 16)[:, :_PIX, :].reshape(n * _PIX, 16)

    y2 = _conv(y1, w2a, b2a, wp=34, relu_to=32)
    c1, ds = _subsample_pad(y2, n, 34, 32)
    y3 = _conv(c1, w22, b22, wp=18, relu_to=32, res=ds, mask=True)

    y4 = _conv(y3, w3a, b3a, wp=18, relu_to=64)
    c1, ds = _subsample_pad(y4, n, 18, 64)
    y5 = _conv(c1, w32, b32, wp=10, relu_to=64, res=ds)

    feats = y5.reshape(n, 10, 10, 64)[:, 1:9, 1:9, :].reshape(n, 64, 64)
    return _pool_fc(feats, fc_w, fc_b)
